# baseline (speedup 1.0000x reference)
import jax
import jax.numpy as jnp
from jax.experimental import pallas as pl

_K = 64
_MAX_ITER = 50


def _kmeans_body(x_ref, p_out_ref, idx_out_ref):
    x = x_ref[...]
    n = x.shape[0]

    x_norm = x / (jnp.sqrt(jnp.sum(x * x, axis=-1, keepdims=True)) + 1e-7)
    sub_iota = jax.lax.broadcasted_iota(jnp.int32, (_K, 1), 0)

    def cond(state):
        _, _, i, done = state
        return jnp.logical_and(i < _MAX_ITER, jnp.logical_not(done))

    n_chunks = 4
    cs = n // n_chunks

    def body(state):
        p, _, i, _ = state
        p_n = p / (jnp.sqrt(jnp.sum(p * p, axis=-1, keepdims=True)) + 1e-7)
        sums = jnp.zeros((_K, x.shape[1]), jnp.float32)
        counts = jnp.zeros((_K, 1), jnp.float32)
        idx_parts = []
        for c in range(n_chunks):
            x_c = x[c * cs:(c + 1) * cs]
            xn_c = x_norm[c * cs:(c + 1) * cs]
            sim = jax.lax.dot_general(
                p_n, xn_c, (((1,), (1,)), ((), ())),
                preferred_element_type=jnp.float32)
            m = jnp.max(sim, axis=0, keepdims=True)
            idx_c = jnp.min(
                jnp.where(sim == m, sub_iota, _K), axis=0, keepdims=True
            ).astype(jnp.int32)
            one_hot = (sub_iota == idx_c).astype(jnp.float32)
            sums = sums + jax.lax.dot_general(
                one_hot, x_c, (((1,), (0,)), ((), ())),
                preferred_element_type=jnp.float32)
            counts = counts + jnp.sum(one_hot, axis=1, keepdims=True)
            idx_parts.append(idx_c)
        idx_new = jnp.concatenate(idx_parts, axis=1)
        p_new = sums / (counts + 1e-6)
        variation = jnp.mean((p_new - p) ** 2)
        return (p_new, idx_new, i + 1, variation < 1e-4)

    p0 = x[:_K]
    idx0 = jnp.zeros((1, n), dtype=jnp.int32)
    p_fin, idx_fin, _, _ = jax.lax.while_loop(
        cond, body, (p0, idx0, jnp.int32(0), jnp.bool_(False)))

    p_out_ref[...] = p_fin
    idx_out_ref[...] = idx_fin


def kernel(x):
    n, d = x.shape
    p, idx = pl.pallas_call(
        _kmeans_body,
        out_shape=(
            jax.ShapeDtypeStruct((_K, d), jnp.float32),
            jax.ShapeDtypeStruct((1, n), jnp.int32),
        ),
    )(x)
    return (p, idx.reshape(n))


# speedup vs baseline: 1.1341x; 1.1341x over previous
import jax
import jax.numpy as jnp
from jax.experimental import pallas as pl

_K = 64
_MAX_ITER = 50


def _iterate(x, x_norm, sub_iota, p, i):
    p_n = p / (jnp.sqrt(jnp.sum(p * p, axis=-1, keepdims=True)) + 1e-7)
    sim = jax.lax.dot_general(
        p_n, x_norm, (((1,), (1,)), ((), ())),
        preferred_element_type=jnp.float32)
    m = jnp.max(sim, axis=0, keepdims=True)
    idx_new = jnp.min(
        jnp.where(sim == m, sub_iota, _K), axis=0, keepdims=True
    ).astype(jnp.int32)
    one_hot = (sub_iota == idx_new).astype(jnp.float32)
    sums = jax.lax.dot_general(
        one_hot, x, (((1,), (0,)), ((), ())),
        preferred_element_type=jnp.float32)
    counts = jnp.sum(one_hot, axis=1, keepdims=True)
    p_new = sums / (counts + 1e-6)
    variation = jnp.mean((p_new - p) ** 2)
    return (p_new, idx_new, i + 1, variation < 1e-4)


def _kmeans_body(x_ref, p_out_ref, idx_out_ref):
    x = x_ref[...]
    n = x.shape[0]

    x_norm = x / (jnp.sqrt(jnp.sum(x * x, axis=-1, keepdims=True)) + 1e-7)
    sub_iota = jax.lax.broadcasted_iota(jnp.int32, (_K, 1), 0)

    def cond(state):
        _, _, i, done = state
        return jnp.logical_and(i < _MAX_ITER, jnp.logical_not(done))

    def body(state):
        p, _, i, _ = state
        return _iterate(x, x_norm, sub_iota, p, i)

    state1 = _iterate(x, x_norm, sub_iota, x[:_K], jnp.int32(0))
    p_fin, idx_fin, _, _ = jax.lax.while_loop(cond, body, state1)

    p_out_ref[...] = p_fin
    idx_out_ref[...] = idx_fin


def kernel(x):
    n, d = x.shape
    p, idx = pl.pallas_call(
        _kmeans_body,
        out_shape=(
            jax.ShapeDtypeStruct((_K, d), jnp.float32),
            jax.ShapeDtypeStruct((1, n), jnp.int32),
        ),
    )(x)
    return (p, idx.reshape(n))


# speedup vs baseline: 1.1796x; 1.0401x over previous
import jax
import jax.numpy as jnp
from jax.experimental import pallas as pl

_K = 64
_MAX_ITER = 50


def _iterate(x, x_norm, sub_iota, p, i):
    p_n = p / (jnp.sqrt(jnp.sum(p * p, axis=-1, keepdims=True)) + 1e-7)
    sim = jax.lax.dot_general(
        p_n, x_norm, (((1,), (1,)), ((), ())),
        preferred_element_type=jnp.float32)
    m = jnp.max(sim, axis=0, keepdims=True)
    idx_f = jnp.min(
        jnp.where(sim == m, sub_iota, float(_K)), axis=0, keepdims=True
    )
    idx_new = idx_f.astype(jnp.int32)
    one_hot = (sub_iota == idx_f).astype(jnp.float32)
    sums = jax.lax.dot_general(
        one_hot, x, (((1,), (0,)), ((), ())),
        preferred_element_type=jnp.float32)
    counts = jnp.sum(one_hot, axis=1, keepdims=True)
    p_new = sums / (counts + 1e-6)
    variation = jnp.mean((p_new - p) ** 2)
    return (p_new, idx_new, i + 1, variation < 1e-4)


def _kmeans_body(x_ref, p_out_ref, idx_out_ref):
    x = x_ref[...]
    n = x.shape[0]

    x_norm = x / (jnp.sqrt(jnp.sum(x * x, axis=-1, keepdims=True)) + 1e-7)
    sub_iota = jax.lax.broadcasted_iota(
        jnp.int32, (_K, 1), 0).astype(jnp.float32)

    def cond(state):
        _, _, i, done = state
        return jnp.logical_and(i < _MAX_ITER, jnp.logical_not(done))

    def body(state):
        p, _, i, _ = state
        return _iterate(x, x_norm, sub_iota, p, i)

    state1 = _iterate(x, x_norm, sub_iota, x[:_K], jnp.int32(0))
    p_fin, idx_fin, _, _ = jax.lax.while_loop(cond, body, state1)

    p_out_ref[...] = p_fin
    idx_out_ref[...] = idx_fin


def kernel(x):
    n, d = x.shape
    p, idx = pl.pallas_call(
        _kmeans_body,
        out_shape=(
            jax.ShapeDtypeStruct((_K, d), jnp.float32),
            jax.ShapeDtypeStruct((1, n), jnp.int32),
        ),
    )(x)
    return (p, idx.reshape(n))


# speedup vs baseline: 1.2159x; 1.0308x over previous
import jax
import jax.numpy as jnp
from jax.experimental import pallas as pl
from jax.experimental.pallas import tpu as pltpu

_K = 64
_MAX_ITER = 50
_COPY_CHUNKS = 4


def _iterate(x, x_norm, sub_iota, p, i):
    p_n = p / (jnp.sqrt(jnp.sum(p * p, axis=-1, keepdims=True)) + 1e-7)
    sim = jax.lax.dot_general(
        p_n, x_norm, (((1,), (1,)), ((), ())),
        preferred_element_type=jnp.float32)
    m = jnp.max(sim, axis=0, keepdims=True)
    idx_f = jnp.min(
        jnp.where(sim == m, sub_iota, float(_K)), axis=0, keepdims=True
    )
    idx_new = idx_f.astype(jnp.int32)
    one_hot = (sub_iota == idx_f).astype(jnp.float32)
    sums = jax.lax.dot_general(
        one_hot, x, (((1,), (0,)), ((), ())),
        preferred_element_type=jnp.float32)
    counts = jnp.sum(one_hot, axis=1, keepdims=True)
    p_new = sums / (counts + 1e-6)
    variation = jnp.mean((p_new - p) ** 2)
    return (p_new, idx_new, i + 1, variation < 1e-4)


def _kmeans_body(x_hbm_ref, p_out_ref, idx_out_ref, x_vmem, sems):
    n = x_hbm_ref.shape[0]
    cs = n // _COPY_CHUNKS

    def _copy(c):
        return pltpu.make_async_copy(
            x_hbm_ref.at[pl.ds(c * cs, cs)],
            x_vmem.at[pl.ds(c * cs, cs)],
            sems.at[c])

    for c in range(_COPY_CHUNKS):
        _copy(c).start()
    x_parts, xn_parts = [], []
    for c in range(_COPY_CHUNKS):
        _copy(c).wait()
        xc = x_vmem[pl.ds(c * cs, cs), :]
        x_parts.append(xc)
        xn_parts.append(
            xc / (jnp.sqrt(jnp.sum(xc * xc, axis=-1, keepdims=True)) + 1e-7))
    x = jnp.concatenate(x_parts, axis=0)
    x_norm = jnp.concatenate(xn_parts, axis=0)
    sub_iota = jax.lax.broadcasted_iota(
        jnp.int32, (_K, 1), 0).astype(jnp.float32)

    def cond(state):
        _, _, i, done = state
        return jnp.logical_and(i < _MAX_ITER, jnp.logical_not(done))

    def body(state):
        p, _, i, _ = state
        return _iterate(x, x_norm, sub_iota, p, i)

    state1 = _iterate(x, x_norm, sub_iota, x[:_K], jnp.int32(0))
    p_fin, idx_fin, _, _ = jax.lax.while_loop(cond, body, state1)

    p_out_ref[...] = p_fin
    idx_out_ref[...] = idx_fin


def kernel(x):
    n, d = x.shape
    p, idx = pl.pallas_call(
        _kmeans_body,
        in_specs=[pl.BlockSpec(memory_space=pl.ANY)],
        out_shape=(
            jax.ShapeDtypeStruct((_K, d), jnp.float32),
            jax.ShapeDtypeStruct((1, n), jnp.int32),
        ),
        scratch_shapes=[
            pltpu.VMEM((n, d), jnp.float32),
            pltpu.SemaphoreType.DMA((_COPY_CHUNKS,)),
        ],
    )(x)
    return (p, idx.reshape(n))
